# baseline (speedup 1.0000x reference)
import functools

import numpy as np
import jax
import jax.numpy as jnp
from jax.experimental import pallas as pl
from jax.experimental.pallas import tpu as pltpu

_NUM_P = 1024
_TINY = np.float32(np.finfo(np.float32).tiny)
_LO_N = np.float32(-0.9999999403953552)
_SQRT2 = np.float32(1.4142135381698608)

_ERFINV_A = [3.43273939e-07, -3.5233877e-06, -4.39150654e-06, 0.00021858087,
             -0.00125372503, -0.00417768164, 0.246640727, 1.50140941]
_ERFINV_B = [0.000100950558, 0.00134934322, -0.00367342844, 0.00573950773,
             -0.0076224613, 0.00943887047, 1.00167406, 2.83297682]


def _np_threefry(k0, k1, x0, x1):
    def rotl(v, r):
        return ((v << np.uint32(r)) | (v >> np.uint32(32 - r))).astype(np.uint32)
    x0 = np.asarray(x0, np.uint32).copy()
    x1 = np.asarray(x1, np.uint32).copy()
    k0 = np.uint32(k0)
    k1 = np.uint32(k1)
    ks2 = np.uint32(k0 ^ k1 ^ np.uint32(0x1BD11BDA))
    ks = [k0, k1, ks2]
    rots = [13, 15, 26, 6, 17, 29, 16, 24]
    x0 = (x0 + k0).astype(np.uint32)
    x1 = (x1 + k1).astype(np.uint32)
    for g in range(5):
        for r in (rots[0:4] if g % 2 == 0 else rots[4:8]):
            x0 = (x0 + x1).astype(np.uint32)
            x1 = rotl(x1, r)
            x1 = (x1 ^ x0).astype(np.uint32)
        x0 = (x0 + ks[(g + 1) % 3]).astype(np.uint32)
        x1 = (x1 + ks[(g + 2) % 3] + np.uint32(g + 1)).astype(np.uint32)
    return x0, x1


def _np_fold_in(key, data):
    o0, o1 = _np_threefry(key[0], key[1], np.array([0], np.uint32),
                          np.array([data], np.uint32))
    return np.array([o0[0], o1[0]], np.uint32)


def _step_keys(T):
    base = np.array([0, 42], np.uint32)
    kn = np.stack([_np_fold_in(base, 2 * t) for t in range(T)])
    kr = np.stack([_np_fold_in(base, 2 * t + 1) for t in range(T)])
    return kn.astype(np.int64).astype(np.int32), kr.astype(np.int64).astype(np.int32)


def _rotl(x, r):
    return jax.lax.shift_left(x, np.int32(r)) | jax.lax.shift_right_logical(
        x, np.int32(32 - r))


def _hash(k0, k1, cnt):
    ks2 = k0 ^ k1 ^ np.int32(0x1BD11BDA)
    ks = (k0, k1, ks2)
    ra = (13, 15, 26, 6)
    rb = (17, 29, 16, 24)
    x0 = jnp.zeros_like(cnt) + k0
    x1 = cnt + k1
    for g in range(5):
        for r in (ra if g % 2 == 0 else rb):
            x0 = x0 + x1
            x1 = _rotl(x1, r)
            x1 = x1 ^ x0
        x0 = x0 + ks[(g + 1) % 3]
        x1 = x1 + ks[(g + 2) % 3] + np.int32(g + 1)
    return x0 ^ x1


def _bits_to_unit(bits):
    m = jax.lax.shift_right_logical(bits, np.int32(9)) | np.int32(0x3F800000)
    return jax.lax.bitcast_convert_type(m, jnp.float32) - np.float32(1.0)


def _erfinv(x):
    w = -jnp.log1p(-x * x)
    wa = w - np.float32(2.5)
    pa = jnp.full_like(x, np.float32(2.81022636e-08))
    for c in _ERFINV_A:
        pa = pa * wa + np.float32(c)
    wb = jnp.sqrt(w) - np.float32(3.0)
    pb = jnp.full_like(x, np.float32(-0.000200214257))
    for c in _ERFINV_B:
        pb = pb * wb + np.float32(c)
    return jnp.where(w < np.float32(5.0), pa, pb) * x


def _pf_kernel(kn_ref, kr_ref, z_ref, obs_ref, out_ref, parts, newp, wts, rws,
               *, P, D, T, PT, KC, PC):
    b = pl.program_id(0)

    parts[...] = jnp.broadcast_to(z_ref[0, 0, :][None, :], (P, D))
    wts[...] = jnp.full((P, 1), np.float32(1.0 / P), jnp.float32)

    iota_nc_p = jax.lax.broadcasted_iota(jnp.int32, (PC, D), 0)
    iota_nc_d = jax.lax.broadcasted_iota(jnp.int32, (PC, D), 1)
    cnt_nc = iota_nc_p * np.int32(D) + iota_nc_d

    iota_kk = jax.lax.broadcasted_iota(jnp.int32, (KC, PT), 0)
    iota_kp = jax.lax.broadcasted_iota(jnp.int32, (KC, PT), 1)
    cnt0 = iota_kp * np.int32(P) + iota_kk
    iota_oh = jax.lax.broadcasted_iota(jnp.int32, (P, PT), 0)

    def step(t, _):
        kn0 = kn_ref[t, 0]
        kn1 = kn_ref[t, 1]
        kr0 = kr_ref[t, 0]
        kr1 = kr_ref[t, 1]

        nbase = b * np.int32(P * D)

        def noise_chunk(c, carry):
            p0 = c * PC
            cnt = nbase + p0 * np.int32(D) + cnt_nc
            f = _bits_to_unit(_hash(kn0, kn1, cnt))
            u = f * np.float32(2.0) + _LO_N
            noise = _SQRT2 * _erfinv(u)
            parts[pl.ds(p0, PC), :] = (parts[pl.ds(p0, PC), :]
                                       + np.float32(0.1) * noise)
            return carry

        jax.lax.fori_loop(0, P // PC, noise_chunk, 0, unroll=False)

        pr = parts[...]
        obs_t = obs_ref[0, t, :][None, :]
        d2 = jnp.sum((pr - obs_t) ** 2, axis=1, keepdims=True)
        lik = jnp.exp(np.float32(-0.5) * d2) + np.float32(1e-8)
        w = wts[...] * lik + np.float32(1e-10)
        w = w / jnp.sum(w)
        wts[...] = w
        rws[...] = np.float32(1.0) / w

        cbase = b * np.int32(P * P)

        def ptile(pt, carry):
            p0 = pt * PT

            def kchunk(kc, acc):
                k0 = kc * KC
                cnt = (cbase + p0 * np.int32(P) + k0) + cnt0
                f = _bits_to_unit(_hash(kr0, kr1, cnt))
                u = f + _TINY
                tv = -jnp.log(u)
                rwc = rws[pl.ds(k0, KC), :]
                val = tv * rwc
                key = (jax.lax.bitcast_convert_type(val, jnp.int32)
                       & np.int32(-1024)) | (k0 + iota_kk)
                return jnp.minimum(acc, key)

            acc0 = jnp.full((KC, PT), np.int32(0x7FFFFFFF), jnp.int32)
            acc = jax.lax.fori_loop(0, P // KC, kchunk, acc0, unroll=False)
            mini = jnp.min(acc, axis=0, keepdims=True) & np.int32(1023)

            onehot = (iota_oh == mini).astype(jnp.float32)
            gathered = jax.lax.dot_general(
                onehot, pr, (((0,), (0,)), ((), ())),
                precision=jax.lax.Precision.HIGHEST,
                preferred_element_type=jnp.float32)
            newp[pl.ds(p0, PT), :] = gathered
            return carry

        jax.lax.fori_loop(0, P // PT, ptile, 0, unroll=False)
        parts[...] = newp[...]
        return _

    jax.lax.fori_loop(0, T, step, 0, unroll=False)
    out_ref[0, 0, :] = jnp.sum(parts[...], axis=0) * np.float32(1.0 / P)


def _build(B, D, T, P, interpret=False):
    PT = min(128, P)
    KC = min(64, P)
    PC = min(64, P)
    grid_spec = pltpu.PrefetchScalarGridSpec(
        num_scalar_prefetch=2,
        grid=(B,),
        in_specs=[
            pl.BlockSpec((1, 1, D), lambda b, *_: (b, 0, 0)),
            pl.BlockSpec((1, T, D), lambda b, *_: (b, 0, 0)),
        ],
        out_specs=pl.BlockSpec((1, 1, D), lambda b, *_: (b, 0, 0)),
        scratch_shapes=[
            pltpu.VMEM((P, D), jnp.float32),
            pltpu.VMEM((P, D), jnp.float32),
            pltpu.VMEM((P, 1), jnp.float32),
            pltpu.VMEM((P, 1), jnp.float32),
        ],
    )
    return pl.pallas_call(
        functools.partial(_pf_kernel, P=P, D=D, T=T, PT=PT, KC=KC, PC=PC),
        grid_spec=grid_spec,
        out_shape=jax.ShapeDtypeStruct((B, 1, D), jnp.float32),
        interpret=interpret,
    )


def _run(z, observation, P, interpret=False):
    B, D = z.shape
    T = observation.shape[2]
    kn, kr = _step_keys(T)
    obs_t = jnp.transpose(observation, (0, 2, 1))
    call = _build(B, D, T, P, interpret=interpret)
    out = call(jnp.asarray(kn), jnp.asarray(kr), z[:, None, :], obs_t)
    return out[:, 0, :]


def kernel(z, observation):
    return _run(z, observation, _NUM_P)


# speedup vs baseline: 1.2357x; 1.2357x over previous
import functools

import numpy as np
import jax
import jax.numpy as jnp
from jax.experimental import pallas as pl
from jax.experimental.pallas import tpu as pltpu

_NUM_P = 1024
_TINY = np.float32(np.finfo(np.float32).tiny)
_LO_N = np.float32(-0.9999999403953552)
_SQRT2 = np.float32(1.4142135381698608)

_ERFINV_A = [3.43273939e-07, -3.5233877e-06, -4.39150654e-06, 0.00021858087,
             -0.00125372503, -0.00417768164, 0.246640727, 1.50140941]
_ERFINV_B = [0.000100950558, 0.00134934322, -0.00367342844, 0.00573950773,
             -0.0076224613, 0.00943887047, 1.00167406, 2.83297682]


def _np_threefry(k0, k1, x0, x1):
    def rotl(v, r):
        return ((v << np.uint32(r)) | (v >> np.uint32(32 - r))).astype(np.uint32)
    x0 = np.asarray(x0, np.uint32).copy()
    x1 = np.asarray(x1, np.uint32).copy()
    k0 = np.uint32(k0)
    k1 = np.uint32(k1)
    ks2 = np.uint32(k0 ^ k1 ^ np.uint32(0x1BD11BDA))
    ks = [k0, k1, ks2]
    rots = [13, 15, 26, 6, 17, 29, 16, 24]
    x0 = (x0 + k0).astype(np.uint32)
    x1 = (x1 + k1).astype(np.uint32)
    for g in range(5):
        for r in (rots[0:4] if g % 2 == 0 else rots[4:8]):
            x0 = (x0 + x1).astype(np.uint32)
            x1 = rotl(x1, r)
            x1 = (x1 ^ x0).astype(np.uint32)
        x0 = (x0 + ks[(g + 1) % 3]).astype(np.uint32)
        x1 = (x1 + ks[(g + 2) % 3] + np.uint32(g + 1)).astype(np.uint32)
    return x0, x1


def _np_fold_in(key, data):
    o0, o1 = _np_threefry(key[0], key[1], np.array([0], np.uint32),
                          np.array([data], np.uint32))
    return np.array([o0[0], o1[0]], np.uint32)


def _step_keys(T):
    base = np.array([0, 42], np.uint32)
    kn = np.stack([_np_fold_in(base, 2 * t) for t in range(T)])
    kr = np.stack([_np_fold_in(base, 2 * t + 1) for t in range(T)])
    return kn.astype(np.int64).astype(np.int32), kr.astype(np.int64).astype(np.int32)


def _rotl(x, r):
    return jax.lax.shift_left(x, np.int32(r)) | jax.lax.shift_right_logical(
        x, np.int32(32 - r))


def _hash(k0, k1, cnt):
    ks2 = k0 ^ k1 ^ np.int32(0x1BD11BDA)
    ks = (k0, k1, ks2)
    ra = (13, 15, 26, 6)
    rb = (17, 29, 16, 24)
    x0 = jnp.zeros_like(cnt) + k0
    x1 = cnt + k1
    for g in range(5):
        for r in (ra if g % 2 == 0 else rb):
            x0 = x0 + x1
            x1 = _rotl(x1, r)
            x1 = x1 ^ x0
        x0 = x0 + ks[(g + 1) % 3]
        x1 = x1 + ks[(g + 2) % 3] + np.int32(g + 1)
    return x0 ^ x1


def _bits_to_unit(bits):
    m = jax.lax.shift_right_logical(bits, np.int32(9)) | np.int32(0x3F800000)
    return jax.lax.bitcast_convert_type(m, jnp.float32) - np.float32(1.0)


def _erfinv(x):
    w = -jnp.log1p(-x * x)
    wa = w - np.float32(2.5)
    pa = jnp.full_like(x, np.float32(2.81022636e-08))
    for c in _ERFINV_A:
        pa = pa * wa + np.float32(c)
    wb = jnp.sqrt(w) - np.float32(3.0)
    pb = jnp.full_like(x, np.float32(-0.000200214257))
    for c in _ERFINV_B:
        pb = pb * wb + np.float32(c)
    return jnp.where(w < np.float32(5.0), pa, pb) * x


def _pf_kernel(kn_ref, kr_ref, z_ref, obs_ref, out_ref, parts, newp, wts, rws,
               *, P, D, T, PT, KC, PC):
    b = pl.program_id(0)

    parts[...] = jnp.broadcast_to(z_ref[0, 0, :][None, :], (P, D))
    wts[...] = jnp.full((P, 1), np.float32(1.0 / P), jnp.float32)

    iota_nc_p = jax.lax.broadcasted_iota(jnp.int32, (PC, D), 0)
    iota_nc_d = jax.lax.broadcasted_iota(jnp.int32, (PC, D), 1)
    cnt_nc = iota_nc_p * np.int32(D) + iota_nc_d

    iota_kk = jax.lax.broadcasted_iota(jnp.int32, (KC, PT), 0)
    iota_kp = jax.lax.broadcasted_iota(jnp.int32, (KC, PT), 1)
    cnt0 = iota_kp * np.int32(P) + iota_kk
    iota_oh = jax.lax.broadcasted_iota(jnp.int32, (P, PT), 0)

    def step(t, _):
        kn0 = kn_ref[t, 0]
        kn1 = kn_ref[t, 1]
        kr0 = kr_ref[t, 0]
        kr1 = kr_ref[t, 1]

        nbase = b * np.int32(P * D)

        def noise_chunk(c, carry):
            p0 = c * PC
            cnt = nbase + p0 * np.int32(D) + cnt_nc
            f = _bits_to_unit(_hash(kn0, kn1, cnt))
            u = f * np.float32(2.0) + _LO_N
            noise = _SQRT2 * _erfinv(u)
            parts[pl.ds(p0, PC), :] = (parts[pl.ds(p0, PC), :]
                                       + np.float32(0.1) * noise)
            return carry

        jax.lax.fori_loop(0, P // PC, noise_chunk, 0, unroll=False)

        pr = parts[...]
        obs_t = obs_ref[0, t, :][None, :]
        d2 = jnp.sum((pr - obs_t) ** 2, axis=1, keepdims=True)
        lik = jnp.exp(np.float32(-0.5) * d2) + np.float32(1e-8)
        w = wts[...] * lik + np.float32(1e-10)
        w = w / jnp.sum(w)
        wts[...] = w
        rws[...] = np.float32(1.0) / w

        cbase = b * np.int32(P * P)

        def ptile(pt, carry):
            p0 = pt * PT

            def kchunk(kc, acc):
                k0 = kc * KC
                cnt = (cbase + p0 * np.int32(P) + k0) + cnt0
                f = _bits_to_unit(_hash(kr0, kr1, cnt))
                rwc = rws[pl.ds(k0, KC), :]
                val = jnp.log(f) * rwc
                key = (jax.lax.bitcast_convert_type(val, jnp.int32)
                       & np.int32(-1024)) | (k0 + iota_kk)
                return jnp.minimum(acc, key)

            acc0 = jnp.full((KC, PT), np.int32(0x7FFFFFFF), jnp.int32)
            acc = jax.lax.fori_loop(0, P // KC, kchunk, acc0, unroll=False)
            mini = jnp.min(acc, axis=0, keepdims=True) & np.int32(1023)

            onehot = (iota_oh == mini).astype(jnp.float32)
            gathered = jax.lax.dot_general(
                onehot, pr, (((0,), (0,)), ((), ())),
                precision=jax.lax.Precision.DEFAULT,
                preferred_element_type=jnp.float32)
            newp[pl.ds(p0, PT), :] = gathered
            return carry

        jax.lax.fori_loop(0, P // PT, ptile, 0, unroll=False)
        parts[...] = newp[...]
        return _

    jax.lax.fori_loop(0, T, step, 0, unroll=False)
    out_ref[0, 0, :] = jnp.sum(parts[...], axis=0) * np.float32(1.0 / P)


def _build(B, D, T, P, interpret=False):
    PT = min(128, P)
    KC = min(128, P)
    PC = min(128, P)
    grid_spec = pltpu.PrefetchScalarGridSpec(
        num_scalar_prefetch=2,
        grid=(B,),
        in_specs=[
            pl.BlockSpec((1, 1, D), lambda b, *_: (b, 0, 0)),
            pl.BlockSpec((1, T, D), lambda b, *_: (b, 0, 0)),
        ],
        out_specs=pl.BlockSpec((1, 1, D), lambda b, *_: (b, 0, 0)),
        scratch_shapes=[
            pltpu.VMEM((P, D), jnp.float32),
            pltpu.VMEM((P, D), jnp.float32),
            pltpu.VMEM((P, 1), jnp.float32),
            pltpu.VMEM((P, 1), jnp.float32),
        ],
    )
    return pl.pallas_call(
        functools.partial(_pf_kernel, P=P, D=D, T=T, PT=PT, KC=KC, PC=PC),
        grid_spec=grid_spec,
        out_shape=jax.ShapeDtypeStruct((B, 1, D), jnp.float32),
        interpret=interpret,
    )


def _run(z, observation, P, interpret=False):
    B, D = z.shape
    T = observation.shape[2]
    kn, kr = _step_keys(T)
    obs_t = jnp.transpose(observation, (0, 2, 1))
    call = _build(B, D, T, P, interpret=interpret)
    out = call(jnp.asarray(kn), jnp.asarray(kr), z[:, None, :], obs_t)
    return out[:, 0, :]


def kernel(z, observation):
    return _run(z, observation, _NUM_P)


# speedup vs baseline: 1.3956x; 1.1293x over previous
import functools

import numpy as np
import jax
import jax.numpy as jnp
from jax.experimental import pallas as pl
from jax.experimental.pallas import tpu as pltpu

_NUM_P = 1024
_TINY = np.float32(np.finfo(np.float32).tiny)
_LO_N = np.float32(-0.9999999403953552)
_SQRT2 = np.float32(1.4142135381698608)

_ERFINV_A = [3.43273939e-07, -3.5233877e-06, -4.39150654e-06, 0.00021858087,
             -0.00125372503, -0.00417768164, 0.246640727, 1.50140941]
_ERFINV_B = [0.000100950558, 0.00134934322, -0.00367342844, 0.00573950773,
             -0.0076224613, 0.00943887047, 1.00167406, 2.83297682]


def _np_threefry(k0, k1, x0, x1):
    def rotl(v, r):
        return ((v << np.uint32(r)) | (v >> np.uint32(32 - r))).astype(np.uint32)
    x0 = np.asarray(x0, np.uint32).copy()
    x1 = np.asarray(x1, np.uint32).copy()
    k0 = np.uint32(k0)
    k1 = np.uint32(k1)
    ks2 = np.uint32(k0 ^ k1 ^ np.uint32(0x1BD11BDA))
    ks = [k0, k1, ks2]
    rots = [13, 15, 26, 6, 17, 29, 16, 24]
    x0 = (x0 + k0).astype(np.uint32)
    x1 = (x1 + k1).astype(np.uint32)
    for g in range(5):
        for r in (rots[0:4] if g % 2 == 0 else rots[4:8]):
            x0 = (x0 + x1).astype(np.uint32)
            x1 = rotl(x1, r)
            x1 = (x1 ^ x0).astype(np.uint32)
        x0 = (x0 + ks[(g + 1) % 3]).astype(np.uint32)
        x1 = (x1 + ks[(g + 2) % 3] + np.uint32(g + 1)).astype(np.uint32)
    return x0, x1


def _np_fold_in(key, data):
    o0, o1 = _np_threefry(key[0], key[1], np.array([0], np.uint32),
                          np.array([data], np.uint32))
    return np.array([o0[0], o1[0]], np.uint32)


def _step_keys(T):
    base = np.array([0, 42], np.uint32)
    kn = np.stack([_np_fold_in(base, 2 * t) for t in range(T)])
    kr = np.stack([_np_fold_in(base, 2 * t + 1) for t in range(T)])
    return kn.astype(np.int64).astype(np.int32), kr.astype(np.int64).astype(np.int32)


def _rotl(x, r):
    return jax.lax.shift_left(x, np.int32(r)) | jax.lax.shift_right_logical(
        x, np.int32(32 - r))


def _hash(k0, k1, cnt):
    ks2 = k0 ^ k1 ^ np.int32(0x1BD11BDA)
    ks = (k0, k1, ks2)
    ra = (13, 15, 26, 6)
    rb = (17, 29, 16, 24)
    x0 = jnp.zeros_like(cnt) + k0
    x1 = cnt + k1
    for g in range(5):
        for r in (ra if g % 2 == 0 else rb):
            x0 = x0 + x1
            x1 = _rotl(x1, r)
            x1 = x1 ^ x0
        x0 = x0 + ks[(g + 1) % 3]
        x1 = x1 + ks[(g + 2) % 3] + np.int32(g + 1)
    return x0 ^ x1


def _bits_to_unit(bits):
    m = jax.lax.shift_right_logical(bits, np.int32(9)) | np.int32(0x3F800000)
    return jax.lax.bitcast_convert_type(m, jnp.float32) - np.float32(1.0)


def _erfinv(x):
    w = -jnp.log1p(-x * x)
    wa = w - np.float32(2.5)
    pa = jnp.full_like(x, np.float32(2.81022636e-08))
    for c in _ERFINV_A:
        pa = pa * wa + np.float32(c)
    wb = jnp.sqrt(w) - np.float32(3.0)
    pb = jnp.full_like(x, np.float32(-0.000200214257))
    for c in _ERFINV_B:
        pb = pb * wb + np.float32(c)
    return jnp.where(w < np.float32(5.0), pa, pb) * x


def _pf_kernel(kn_ref, kr_ref, z_ref, obs_ref, out_ref, parts, newp, wts, rws,
               *, P, D, T, PT, KC, PC):
    b = pl.program_id(0)

    parts[...] = jnp.broadcast_to(z_ref[0, 0, :][None, :], (P, D))
    wts[...] = jnp.full((P, 1), np.float32(1.0 / P), jnp.float32)

    iota_nc_p = jax.lax.broadcasted_iota(jnp.int32, (PC, D), 0)
    iota_nc_d = jax.lax.broadcasted_iota(jnp.int32, (PC, D), 1)
    cnt_nc = iota_nc_p * np.int32(D) + iota_nc_d

    iota_kk = jax.lax.broadcasted_iota(jnp.int32, (KC, PT), 0)
    iota_kp = jax.lax.broadcasted_iota(jnp.int32, (KC, PT), 1)
    cnt0 = iota_kp * np.int32(P) + iota_kk
    iota_oh = jax.lax.broadcasted_iota(jnp.int32, (P, PT), 0)

    def step(t, _):
        kn0 = kn_ref[t, 0]
        kn1 = kn_ref[t, 1]
        kr0 = kr_ref[t, 0]
        kr1 = kr_ref[t, 1]

        nbase = b * np.int32(P * D)

        for c in range(P // PC):
            p0 = c * PC
            cnt = nbase + np.int32(p0 * D) + cnt_nc
            f = _bits_to_unit(_hash(kn0, kn1, cnt))
            u = f * np.float32(2.0) + _LO_N
            noise = _SQRT2 * _erfinv(u)
            parts[p0:p0 + PC, :] = parts[p0:p0 + PC, :] + np.float32(0.1) * noise

        pr = parts[...]
        obs_t = obs_ref[0, t, :][None, :]
        d2 = jnp.sum((pr - obs_t) ** 2, axis=1, keepdims=True)
        lik = jnp.exp(np.float32(-0.5) * d2) + np.float32(1e-8)
        w = wts[...] * lik + np.float32(1e-10)
        w = w / jnp.sum(w)
        wts[...] = w
        rws[...] = np.float32(1.0) / w

        cbase = b * np.int32(P * P)

        def ptile(pt, carry):
            p0 = pt * PT

            acc = jnp.full((KC, PT), np.int32(0x7FFFFFFF), jnp.int32)
            for kc in range(P // KC):
                k0 = kc * KC
                cnt = (cbase + p0 * np.int32(P) + np.int32(k0)) + cnt0
                f = _bits_to_unit(_hash(kr0, kr1, cnt))
                rwc = rws[k0:k0 + KC, :]
                val = jnp.log(f) * rwc
                key = (jax.lax.bitcast_convert_type(val, jnp.int32)
                       & np.int32(-1024)) | np.int32(k0) + iota_kk
                acc = jnp.minimum(acc, key)
            mini = jnp.min(acc, axis=0, keepdims=True) & np.int32(1023)

            onehot = (iota_oh == mini).astype(jnp.float32)
            gathered = jax.lax.dot_general(
                onehot, pr, (((0,), (0,)), ((), ())),
                precision=jax.lax.Precision.DEFAULT,
                preferred_element_type=jnp.float32)
            newp[pl.ds(p0, PT), :] = gathered
            return carry

        jax.lax.fori_loop(0, P // PT, ptile, 0, unroll=False)
        parts[...] = newp[...]
        return _

    jax.lax.fori_loop(0, T, step, 0, unroll=False)
    out_ref[0, 0, :] = jnp.sum(parts[...], axis=0) * np.float32(1.0 / P)


def _build(B, D, T, P, interpret=False):
    PT = min(128, P)
    KC = min(128, P)
    PC = min(128, P)
    grid_spec = pltpu.PrefetchScalarGridSpec(
        num_scalar_prefetch=2,
        grid=(B,),
        in_specs=[
            pl.BlockSpec((1, 1, D), lambda b, *_: (b, 0, 0)),
            pl.BlockSpec((1, T, D), lambda b, *_: (b, 0, 0)),
        ],
        out_specs=pl.BlockSpec((1, 1, D), lambda b, *_: (b, 0, 0)),
        scratch_shapes=[
            pltpu.VMEM((P, D), jnp.float32),
            pltpu.VMEM((P, D), jnp.float32),
            pltpu.VMEM((P, 1), jnp.float32),
            pltpu.VMEM((P, 1), jnp.float32),
        ],
    )
    return pl.pallas_call(
        functools.partial(_pf_kernel, P=P, D=D, T=T, PT=PT, KC=KC, PC=PC),
        grid_spec=grid_spec,
        out_shape=jax.ShapeDtypeStruct((B, 1, D), jnp.float32),
        interpret=interpret,
    )


def _run(z, observation, P, interpret=False):
    B, D = z.shape
    T = observation.shape[2]
    kn, kr = _step_keys(T)
    obs_t = jnp.transpose(observation, (0, 2, 1))
    call = _build(B, D, T, P, interpret=interpret)
    out = call(jnp.asarray(kn), jnp.asarray(kr), z[:, None, :], obs_t)
    return out[:, 0, :]


def kernel(z, observation):
    return _run(z, observation, _NUM_P)


# speedup vs baseline: 1.5913x; 1.1403x over previous
import functools

import numpy as np
import jax
import jax.numpy as jnp
from jax.experimental import pallas as pl
from jax.experimental.pallas import tpu as pltpu

_NUM_P = 1024
_LO_N = np.float32(-0.9999999403953552)
_SQRT2 = np.float32(1.4142135381698608)

_ERFINV_A = [3.43273939e-07, -3.5233877e-06, -4.39150654e-06, 0.00021858087,
             -0.00125372503, -0.00417768164, 0.246640727, 1.50140941]
_ERFINV_B = [0.000100950558, 0.00134934322, -0.00367342844, 0.00573950773,
             -0.0076224613, 0.00943887047, 1.00167406, 2.83297682]


def _np_threefry(k0, k1, x0, x1):
    def rotl(v, r):
        return ((v << np.uint32(r)) | (v >> np.uint32(32 - r))).astype(np.uint32)
    x0 = np.asarray(x0, np.uint32).copy()
    x1 = np.asarray(x1, np.uint32).copy()
    k0 = np.uint32(k0)
    k1 = np.uint32(k1)
    ks2 = np.uint32(k0 ^ k1 ^ np.uint32(0x1BD11BDA))
    ks = [k0, k1, ks2]
    rots = [13, 15, 26, 6, 17, 29, 16, 24]
    x0 = (x0 + k0).astype(np.uint32)
    x1 = (x1 + k1).astype(np.uint32)
    for g in range(5):
        for r in (rots[0:4] if g % 2 == 0 else rots[4:8]):
            x0 = (x0 + x1).astype(np.uint32)
            x1 = rotl(x1, r)
            x1 = (x1 ^ x0).astype(np.uint32)
        x0 = (x0 + ks[(g + 1) % 3]).astype(np.uint32)
        x1 = (x1 + ks[(g + 2) % 3] + np.uint32(g + 1)).astype(np.uint32)
    return x0, x1


def _np_fold_in(key, data):
    o0, o1 = _np_threefry(key[0], key[1], np.array([0], np.uint32),
                          np.array([data], np.uint32))
    return np.array([o0[0], o1[0]], np.uint32)


def _step_keys(T):
    base = np.array([0, 42], np.uint32)
    kn = np.stack([_np_fold_in(base, 2 * t) for t in range(T)])
    kr = np.stack([_np_fold_in(base, 2 * t + 1) for t in range(T)])
    return kn.astype(np.int64).astype(np.int32), kr.astype(np.int64).astype(np.int32)


def _rotl(x, r):
    return jax.lax.shift_left(x, np.int32(r)) | jax.lax.shift_right_logical(
        x, np.int32(32 - r))


def _hash(k0, k1, cnt):
    ks2 = k0 ^ k1 ^ np.int32(0x1BD11BDA)
    ks = (k0, k1, ks2)
    ra = (13, 15, 26, 6)
    rb = (17, 29, 16, 24)
    x1 = cnt + k1
    x0 = x1 + k0
    x1 = _rotl(x1, 13) ^ x0
    for r in (15, 26, 6):
        x0 = x0 + x1
        x1 = _rotl(x1, r)
        x1 = x1 ^ x0
    x0 = x0 + ks[1]
    x1 = x1 + (ks[2] + np.int32(1))
    for g in range(1, 5):
        for r in (ra if g % 2 == 0 else rb):
            x0 = x0 + x1
            x1 = _rotl(x1, r)
            x1 = x1 ^ x0
        x0 = x0 + ks[(g + 1) % 3]
        x1 = x1 + (ks[(g + 2) % 3] + np.int32(g + 1))
    return x0 ^ x1


def _bits_to_unit(bits):
    m = jax.lax.shift_right_logical(bits, np.int32(9)) | np.int32(0x3F800000)
    return jax.lax.bitcast_convert_type(m, jnp.float32) - np.float32(1.0)


def _erfinv(x):
    w = -jnp.log1p(-x * x)
    wa = w - np.float32(2.5)
    pa = jnp.full_like(x, np.float32(2.81022636e-08))
    for c in _ERFINV_A:
        pa = pa * wa + np.float32(c)
    wb = jnp.sqrt(w) - np.float32(3.0)
    pb = jnp.full_like(x, np.float32(-0.000200214257))
    for c in _ERFINV_B:
        pb = pb * wb + np.float32(c)
    return jnp.where(w < np.float32(5.0), pa, pb) * x


def _pf_kernel(kn_ref, kr_ref, z_ref, obs_ref, out_ref, parts, newp, wts,
               minis, ident, *, P, D, T, PP, KL):
    b = pl.program_id(0)
    NKC = P // KL
    NPT = P // PP

    parts[...] = jnp.broadcast_to(z_ref[0], (D, P))
    wts[...] = jnp.full((1, P), np.float32(1.0 / P), jnp.float32)
    ident[...] = (jax.lax.broadcasted_iota(jnp.int32, (P, P), 0)
                  == jax.lax.broadcasted_iota(jnp.int32, (P, P), 1)
                  ).astype(jnp.float32)

    iota_nd = jax.lax.broadcasted_iota(jnp.int32, (D, KL), 0)
    iota_np = jax.lax.broadcasted_iota(jnp.int32, (D, KL), 1)
    cnt_n0 = iota_np * np.int32(D) + iota_nd

    iota_pp = jax.lax.broadcasted_iota(jnp.int32, (PP, KL), 0)
    iota_kk = jax.lax.broadcasted_iota(jnp.int32, (PP, KL), 1)
    cnt_c0 = iota_pp * np.int32(P) + iota_kk

    iota_ohk = jax.lax.broadcasted_iota(jnp.int32, (P, P), 0)

    def step(t, carry):
        kn0 = kn_ref[t, 0]
        kn1 = kn_ref[t, 1]
        kr0 = kr_ref[t, 0]
        kr1 = kr_ref[t, 1]

        nbase = b * np.int32(P * D)
        for c in range(P // KL):
            cnt = (nbase + np.int32(c * KL * D)) + cnt_n0
            f = _bits_to_unit(_hash(kn0, kn1, cnt))
            u = f * np.float32(2.0) + _LO_N
            noise = _SQRT2 * _erfinv(u)
            sl = slice(c * KL, (c + 1) * KL)
            parts[:, sl] = parts[:, sl] + np.float32(0.1) * noise

        pr = parts[...]
        obs_t = obs_ref[0, t]
        d2 = jnp.sum((pr - obs_t) ** 2, axis=0, keepdims=True)
        lik = jnp.exp(np.float32(-0.5) * d2) + np.float32(1e-8)
        w = wts[...] * lik + np.float32(1e-10)
        w = w / jnp.sum(w)
        wts[...] = w
        rw = np.float32(1.0) / w

        cbase = b * np.int32(P * P)

        def ptile(pt, tc):
            p0 = pt * PP
            acc_b = jnp.full((PP, KL), np.int32(0x7FFFFFFF), jnp.int32)
            acc_k = jnp.zeros((PP, KL), jnp.int32)
            for kc in range(NKC):
                k0 = kc * KL
                cnt = (cbase + p0 * np.int32(P) + np.int32(k0)) + cnt_c0
                f = _bits_to_unit(_hash(kr0, kr1, cnt))
                rwc = jax.lax.slice(rw, (0, k0), (1, k0 + KL))
                val = jnp.log(f) * rwc
                vb = jax.lax.bitcast_convert_type(val, jnp.int32)
                better = vb < acc_b
                acc_b = jnp.minimum(acc_b, vb)
                acc_k = jnp.where(better, np.int32(k0) + iota_kk, acc_k)
            mv = jnp.min(acc_b, axis=1, keepdims=True)
            cand = jnp.where(acc_b == mv, acc_k, np.int32(2**30))
            mini = jnp.min(cand, axis=1, keepdims=True)
            minis[pl.ds(p0, PP), :] = mini.astype(jnp.float32)
            return tc

        jax.lax.fori_loop(0, NPT, ptile, 0, unroll=False)

        mrow = jax.lax.dot_general(
            minis[...], ident[...], (((0,), (0,)), ((), ())),
            precision=jax.lax.Precision.DEFAULT,
            preferred_element_type=jnp.float32)
        onehot = (iota_ohk == mrow.astype(jnp.int32)).astype(jnp.float32)
        newp[...] = jax.lax.dot_general(
            pr, onehot, (((1,), (0,)), ((), ())),
            precision=jax.lax.Precision.DEFAULT,
            preferred_element_type=jnp.float32)
        parts[...] = newp[...]
        return carry

    jax.lax.fori_loop(0, T, step, 0, unroll=False)
    out_ref[0] = jnp.sum(parts[...], axis=1, keepdims=True) * np.float32(1.0 / P)


def _build(B, D, T, P, interpret=False):
    PP = min(128, P)
    KL = min(128, P)
    grid_spec = pltpu.PrefetchScalarGridSpec(
        num_scalar_prefetch=2,
        grid=(B,),
        in_specs=[
            pl.BlockSpec((1, D, 1), lambda b, *_: (b, 0, 0)),
            pl.BlockSpec((1, T, D, 1), lambda b, *_: (b, 0, 0, 0)),
        ],
        out_specs=pl.BlockSpec((1, D, 1), lambda b, *_: (b, 0, 0)),
        scratch_shapes=[
            pltpu.VMEM((D, P), jnp.float32),
            pltpu.VMEM((D, P), jnp.float32),
            pltpu.VMEM((1, P), jnp.float32),
            pltpu.VMEM((P, 1), jnp.float32),
            pltpu.VMEM((P, P), jnp.float32),
        ],
    )
    return pl.pallas_call(
        functools.partial(_pf_kernel, P=P, D=D, T=T, PP=PP, KL=KL),
        grid_spec=grid_spec,
        out_shape=jax.ShapeDtypeStruct((B, D, 1), jnp.float32),
        interpret=interpret,
    )


def _run(z, observation, P, interpret=False):
    B, D = z.shape
    T = observation.shape[2]
    kn, kr = _step_keys(T)
    obs_t = jnp.transpose(observation, (0, 2, 1))[:, :, :, None]
    call = _build(B, D, T, P, interpret=interpret)
    out = call(jnp.asarray(kn), jnp.asarray(kr), z[:, :, None], obs_t)
    return out[:, :, 0]


def kernel(z, observation):
    return _run(z, observation, _NUM_P)
